# baseline (device time: 13975 ns/iter reference)
import jax
import jax.numpy as jnp
from jax import lax
from jax.experimental import pallas as pl
from jax.experimental.pallas import tpu as pltpu

N_DEV = 4
C = 4


def kernel(t, W):
    m, k = t.shape
    _, n = W.shape
    mc = m // C

    def body(t_ref, w_ref, out_ref, comm_ref, send_sems, recv_sems):
        my = lax.axis_index("i")
        p1 = my ^ 1
        p2 = my ^ 2

        barrier_sem = pltpu.get_barrier_semaphore()
        for nbr in (p1, p2):
            pl.semaphore_signal(
                barrier_sem, inc=1,
                device_id=(nbr,), device_id_type=pl.DeviceIdType.MESH,
            )

        y = jnp.dot(
            t_ref[...].astype(jnp.bfloat16),
            w_ref[...].astype(jnp.bfloat16),
            preferred_element_type=jnp.float32,
        )
        comm_ref[0] = y.astype(jnp.bfloat16).reshape(C, mc, n)

        pl.semaphore_wait(barrier_sem, 2)

        r1 = [
            pltpu.make_async_remote_copy(
                src_ref=comm_ref.at[0, c],
                dst_ref=comm_ref.at[1, c],
                send_sem=send_sems.at[0, c],
                recv_sem=recv_sems.at[0, c],
                device_id=(p1,),
                device_id_type=pl.DeviceIdType.MESH,
            )
            for c in range(C)
        ]
        r2 = [
            pltpu.make_async_remote_copy(
                src_ref=comm_ref.at[2, c],
                dst_ref=comm_ref.at[3, c],
                send_sem=send_sems.at[1, c],
                recv_sem=recv_sems.at[1, c],
                device_id=(p2,),
                device_id_type=pl.DeviceIdType.MESH,
            )
            for c in range(C)
        ]
        for c in range(C):
            r1[c].start()
        for c in range(C):
            r1[c].wait_recv()
            s2c = y[c * mc:(c + 1) * mc] + comm_ref[1, c].astype(jnp.float32)
            comm_ref[2, c] = s2c.astype(jnp.bfloat16)
            r2[c].start()
        for c in range(C):
            r2[c].wait_recv()
            out_ref[c * mc:(c + 1) * mc, :] = (
                comm_ref[2, c].astype(jnp.float32)
                + comm_ref[3, c].astype(jnp.float32)
            )
        for c in range(C):
            r1[c].wait_send()
            r2[c].wait_send()

    return pl.pallas_call(
        body,
        out_shape=jax.ShapeDtypeStruct((m, n), jnp.float32),
        in_specs=[
            pl.BlockSpec(memory_space=pltpu.VMEM),
            pl.BlockSpec(memory_space=pltpu.VMEM),
        ],
        out_specs=pl.BlockSpec(memory_space=pltpu.VMEM),
        scratch_shapes=[
            pltpu.VMEM((4, C, mc, n), jnp.bfloat16),
            pltpu.SemaphoreType.DMA((2, C)),
            pltpu.SemaphoreType.DMA((2, C)),
        ],
        compiler_params=pltpu.CompilerParams(collective_id=0),
    )(t, W)
